# baseline (speedup 1.0000x reference)
import functools

import jax
import jax.numpy as jnp
from jax import lax
from jax.experimental import pallas as pl
from jax.experimental.pallas import tpu as pltpu
from jax.experimental.pallas import tpu_sc as plsc

L = 128
NS = 16
IB = 16
NB = 2


def _sc_aggregate(n_pad, cpt, x, src_p, dst_p, init0):
    d = x.shape[1]
    rows_per_tile = n_pad // NS

    mesh = plsc.VectorSubcoreMesh(core_axis_name="c", subcore_axis_name="s")

    @functools.partial(
        pl.kernel,
        out_type=jax.ShapeDtypeStruct((n_pad, d), jnp.float32),
        mesh=mesh,
        scratch_types=[
            pltpu.VMEM_SHARED((n_pad, d), jnp.float32),
            pltpu.VMEM((IB, L), jnp.int32),
            pltpu.VMEM((IB, L), jnp.int32),
        ]
        + [pltpu.VMEM((L, d), jnp.float32)] * NB
        + [pltpu.SemaphoreType.DMA] * (2 * NB),
    )
    def agg(x_hbm, src_hbm, dst_hbm, init0_hbm, p0_hbm,
            acc, src_idx, dst_idx, *bufs_sems):
        rows = bufs_sems[:NB]
        gsem = bufs_sems[NB:2 * NB]
        ssem = bufs_sems[2 * NB:]
        cid = lax.axis_index("c")
        sid = lax.axis_index("s")
        chunk_off = sid * cpt
        n_blocks = lax.select(cid == 0, cpt // IB, 0)

        ibase = sid * rows_per_tile

        @pl.when(cid == 0)
        def _():
            pltpu.sync_copy(init0_hbm.at[pl.ds(ibase, rows_per_tile)],
                            acc.at[pl.ds(ibase, rows_per_tile)])

        plsc.subcore_barrier()

        def gissue(j, b):
            pltpu.async_copy(x_hbm.at[src_idx.at[j]], rows[b], gsem[b])

        def gwait(b):
            pltpu.make_async_copy(
                x_hbm.at[src_idx.at[0]], rows[b], gsem[b]).wait()

        def sissue(j, b):
            pltpu.async_copy(rows[b], acc.at[dst_idx.at[j]], ssem[b],
                             add=True)

        def swait(b):
            pltpu.make_async_copy(
                rows[b], acc.at[dst_idx.at[0]], ssem[b]).wait()

        def blk_body(bi, carry):
            base = chunk_off + bi * IB
            pltpu.sync_copy(src_hbm.at[pl.ds(base, IB)], src_idx)
            pltpu.sync_copy(dst_hbm.at[pl.ds(base, IB)], dst_idx)
            for b in range(NB):
                gissue(b, b)

            def grp(g, c):
                j0 = g * NB
                for b in range(NB):
                    gwait(b)
                    sissue(j0 + b, b)
                for b in range(NB):
                    @pl.when(j0 + NB + b < IB)
                    def _(b=b):
                        swait(b)
                        gissue(j0 + NB + b, b)
                return c

            carry = lax.fori_loop(0, IB // NB, grp, carry)
            for b in range(NB):
                swait(b)
            return carry

        lax.fori_loop(0, n_blocks, blk_body, 0)
        plsc.subcore_barrier()

        @pl.when(cid == 0)
        def _():
            pltpu.sync_copy(acc.at[pl.ds(ibase, rows_per_tile)],
                            p0_hbm.at[pl.ds(ibase, rows_per_tile)])

    return agg(x, src_p, dst_p, init0)


def _mlp_bn_body(p0, w1, b1, w2, b2, w3, b3, gamma, beta, out):
    n = out.shape[0]
    h = p0[pl.ds(0, n), :]
    a1 = jax.nn.sigmoid(
        jnp.dot(h, w1[...], preferred_element_type=jnp.float32) + b1[...])
    a2 = jax.nn.sigmoid(
        jnp.dot(a1, w2[...], preferred_element_type=jnp.float32) + b2[...])
    a3 = jnp.dot(a2, w3[...], preferred_element_type=jnp.float32) + b3[...]
    act = jnp.where(a3 >= 0.0, a3, 0.01 * a3)
    mean = jnp.sum(act, axis=0, keepdims=True) / n
    cent = act - mean
    var = jnp.sum(cent * cent, axis=0, keepdims=True) / n
    out[...] = cent * lax.rsqrt(var + 1e-5) * gamma[...] + beta[...]


def kernel(x, edge_index, W1, b1, W2, b2, W3, b3, gamma, beta):
    n, d = x.shape
    e = edge_index.shape[1]

    chunks = -(-e // L)
    cpt = IB * (-(-chunks // (IB * NS)))
    c_pad = NS * cpt
    e_pad = c_pad * L
    n_pad = (-(-(n + 1) // 128)) * 128

    src = edge_index[0]
    dst = edge_index[1]
    pad = e_pad - e
    src_p = jnp.concatenate([src, jnp.zeros((pad,), jnp.int32)]).reshape(c_pad, L)
    dump = n + jnp.arange(pad, dtype=jnp.int32) % (n_pad - n)
    dst_p = jnp.concatenate([dst, dump]).reshape(c_pad, L)
    init0 = jnp.concatenate([x, jnp.zeros((n_pad - n, d), jnp.float32)], axis=0)

    p0 = _sc_aggregate(n_pad, cpt, x, src_p, dst_p, init0)

    h = pl.pallas_call(
        _mlp_bn_body,
        out_shape=jax.ShapeDtypeStruct((n, d), jnp.float32),
    )(p0, W1, b1.reshape(1, -1), W2, b2.reshape(1, -1),
      W3, b3.reshape(1, -1), gamma.reshape(1, -1), beta.reshape(1, -1))

    return (h, edge_index)


# speedup vs baseline: 3.0857x; 3.0857x over previous
import functools

import jax
import jax.numpy as jnp
from jax import lax
from jax.experimental import pallas as pl
from jax.experimental.pallas import tpu as pltpu
from jax.experimental.pallas import tpu_sc as plsc

L = 128
NW = 32
NS = 16
IB = 16
NB = 2


def _sc_aggregate(n_pad, cpt, x, src_p, dst_p, init0, init1):
    d = x.shape[1]
    rows_per_tile = n_pad // NS

    mesh = plsc.VectorSubcoreMesh(core_axis_name="c", subcore_axis_name="s")

    @functools.partial(
        pl.kernel,
        out_type=(
            jax.ShapeDtypeStruct((n_pad, d), jnp.float32),
            jax.ShapeDtypeStruct((n_pad, d), jnp.float32),
        ),
        mesh=mesh,
        scratch_types=[
            pltpu.VMEM_SHARED((n_pad, d), jnp.float32),
            pltpu.VMEM((IB, L), jnp.int32),
            pltpu.VMEM((IB, L), jnp.int32),
        ]
        + [pltpu.VMEM((L, d), jnp.float32)] * NB
        + [pltpu.SemaphoreType.DMA] * (2 * NB),
    )
    def agg(x_hbm, src_hbm, dst_hbm, init0_hbm, init1_hbm, p0_hbm, p1_hbm,
            acc, src_idx, dst_idx, *bufs_sems):
        rows = bufs_sems[:NB]
        gsem = bufs_sems[NB:2 * NB]
        ssem = bufs_sems[2 * NB:]
        cid = lax.axis_index("c")
        sid = lax.axis_index("s")
        wid = sid * 2 + cid

        ibase = sid * rows_per_tile

        @pl.when(cid == 0)
        def _():
            pltpu.sync_copy(init0_hbm.at[pl.ds(ibase, rows_per_tile)],
                            acc.at[pl.ds(ibase, rows_per_tile)])

        @pl.when(cid != 0)
        def _():
            pltpu.sync_copy(init1_hbm.at[pl.ds(ibase, rows_per_tile)],
                            acc.at[pl.ds(ibase, rows_per_tile)])

        plsc.subcore_barrier()

        def gissue(j, b):
            pltpu.async_copy(x_hbm.at[src_idx.at[j]], rows[b], gsem[b])

        def gwait(b):
            pltpu.make_async_copy(
                x_hbm.at[src_idx.at[0]], rows[b], gsem[b]).wait()

        def sissue(j, b):
            pltpu.async_copy(rows[b], acc.at[dst_idx.at[j]], ssem[b],
                             add=True)

        def swait(b):
            pltpu.make_async_copy(
                rows[b], acc.at[dst_idx.at[0]], ssem[b]).wait()

        def blk_body(bi, carry):
            pltpu.sync_copy(src_hbm.at[wid, pl.ds(bi * IB, IB)], src_idx)
            pltpu.sync_copy(dst_hbm.at[wid, pl.ds(bi * IB, IB)], dst_idx)
            for b in range(NB):
                gissue(b, b)

            def grp(g, c):
                j0 = g * NB
                for b in range(NB):
                    gwait(b)
                    sissue(j0 + b, b)
                for b in range(NB):
                    @pl.when(j0 + NB + b < IB)
                    def _(b=b):
                        swait(b)
                        gissue(j0 + NB + b, b)
                return c

            carry = lax.fori_loop(0, IB // NB, grp, carry)
            for b in range(NB):
                swait(b)
            return carry

        lax.fori_loop(0, cpt // IB, blk_body, 0)
        plsc.subcore_barrier()

        @pl.when(cid == 0)
        def _():
            pltpu.sync_copy(acc.at[pl.ds(ibase, rows_per_tile)],
                            p0_hbm.at[pl.ds(ibase, rows_per_tile)])

        @pl.when(cid != 0)
        def _():
            pltpu.sync_copy(acc.at[pl.ds(ibase, rows_per_tile)],
                            p1_hbm.at[pl.ds(ibase, rows_per_tile)])

    return agg(x, src_p, dst_p, init0, init1)


def _mlp_bn_body(p0, p1, w1, b1, w2, b2, w3, b3, gamma, beta, out):
    n = out.shape[0]
    h = p0[pl.ds(0, n), :] + p1[pl.ds(0, n), :]
    a1 = jax.nn.sigmoid(
        jnp.dot(h, w1[...], preferred_element_type=jnp.float32) + b1[...])
    a2 = jax.nn.sigmoid(
        jnp.dot(a1, w2[...], preferred_element_type=jnp.float32) + b2[...])
    a3 = jnp.dot(a2, w3[...], preferred_element_type=jnp.float32) + b3[...]
    act = jnp.where(a3 >= 0.0, a3, 0.01 * a3)
    mean = jnp.sum(act, axis=0, keepdims=True) / n
    cent = act - mean
    var = jnp.sum(cent * cent, axis=0, keepdims=True) / n
    out[...] = cent * lax.rsqrt(var + 1e-5) * gamma[...] + beta[...]


def kernel(x, edge_index, W1, b1, W2, b2, W3, b3, gamma, beta):
    n, d = x.shape
    e = edge_index.shape[1]

    chunks = -(-e // L)
    cpt = IB * (-(-chunks // (IB * NW)))
    e_pad = NW * cpt * L
    n_pad = (-(-(n + 1) // 128)) * 128

    src = edge_index[0]
    dst = edge_index[1]
    pad = e_pad - e
    iota = jnp.arange(pad, dtype=jnp.int32)
    src_p = jnp.concatenate([src, iota % n]).reshape(NW, cpt, L)
    dst_p = jnp.concatenate([dst, n + iota % (n_pad - n)]).reshape(NW, cpt, L)
    init0 = jnp.concatenate([x, jnp.zeros((n_pad - n, d), jnp.float32)], axis=0)
    init1 = jnp.zeros((n_pad, d), jnp.float32)

    p0, p1 = _sc_aggregate(n_pad, cpt, x, src_p, dst_p, init0, init1)

    h = pl.pallas_call(
        _mlp_bn_body,
        out_shape=jax.ShapeDtypeStruct((n, d), jnp.float32),
    )(p0, p1, W1, b1.reshape(1, -1), W2, b2.reshape(1, -1),
      W3, b3.reshape(1, -1), gamma.reshape(1, -1), beta.reshape(1, -1))

    return (h, edge_index)


# speedup vs baseline: 3.1619x; 1.0247x over previous
import functools

import jax
import jax.numpy as jnp
from jax import lax
from jax.experimental import pallas as pl
from jax.experimental.pallas import tpu as pltpu
from jax.experimental.pallas import tpu_sc as plsc

L = 128
NW = 32
NS = 16
IB = 40
NB = 2


def _sc_aggregate(n_pad, cpt, x, src_p, dst_p, init0, init1):
    d = x.shape[1]
    rows_per_tile = n_pad // NS

    mesh = plsc.VectorSubcoreMesh(core_axis_name="c", subcore_axis_name="s")

    @functools.partial(
        pl.kernel,
        out_type=(
            jax.ShapeDtypeStruct((n_pad, d), jnp.float32),
            jax.ShapeDtypeStruct((n_pad, d), jnp.float32),
        ),
        mesh=mesh,
        scratch_types=[
            pltpu.VMEM_SHARED((n_pad, d), jnp.float32),
            pltpu.VMEM((IB, L), jnp.int32),
            pltpu.VMEM((IB, L), jnp.int32),
        ]
        + [pltpu.VMEM((L, d), jnp.float32)] * NB
        + [pltpu.SemaphoreType.DMA] * (2 * NB),
    )
    def agg(x_hbm, src_hbm, dst_hbm, init0_hbm, init1_hbm, p0_hbm, p1_hbm,
            acc, src_idx, dst_idx, *bufs_sems):
        rows = bufs_sems[:NB]
        gsem = bufs_sems[NB:2 * NB]
        ssem = bufs_sems[2 * NB:]
        cid = lax.axis_index("c")
        sid = lax.axis_index("s")
        wid = sid * 2 + cid

        ibase = sid * rows_per_tile

        @pl.when(cid == 0)
        def _():
            pltpu.sync_copy(init0_hbm.at[pl.ds(ibase, rows_per_tile)],
                            acc.at[pl.ds(ibase, rows_per_tile)])

        @pl.when(cid != 0)
        def _():
            pltpu.sync_copy(init1_hbm.at[pl.ds(ibase, rows_per_tile)],
                            acc.at[pl.ds(ibase, rows_per_tile)])

        plsc.subcore_barrier()

        def gissue(j, b):
            pltpu.async_copy(x_hbm.at[src_idx.at[j]], rows[b], gsem[b])

        def gwait(b):
            pltpu.make_async_copy(
                x_hbm.at[src_idx.at[0]], rows[b], gsem[b]).wait()

        def sissue(j, b):
            pltpu.async_copy(rows[b], acc.at[dst_idx.at[j]], ssem[b],
                             add=True)

        def swait(b):
            pltpu.make_async_copy(
                rows[b], acc.at[dst_idx.at[0]], ssem[b]).wait()

        def blk_body(bi, carry):
            pltpu.sync_copy(src_hbm.at[wid, pl.ds(bi * IB, IB)], src_idx)
            pltpu.sync_copy(dst_hbm.at[wid, pl.ds(bi * IB, IB)], dst_idx)
            for b in range(NB):
                gissue(b, b)

            def grp(g, c):
                j0 = g * NB
                for b in range(NB):
                    gwait(b)
                    sissue(j0 + b, b)
                for b in range(NB):
                    @pl.when(j0 + NB + b < IB)
                    def _(b=b):
                        swait(b)
                        gissue(j0 + NB + b, b)
                return c

            carry = lax.fori_loop(0, IB // NB, grp, carry)
            for b in range(NB):
                swait(b)
            return carry

        lax.fori_loop(0, cpt // IB, blk_body, 0)
        plsc.subcore_barrier()

        @pl.when(cid == 0)
        def _():
            pltpu.sync_copy(acc.at[pl.ds(ibase, rows_per_tile)],
                            p0_hbm.at[pl.ds(ibase, rows_per_tile)])

        @pl.when(cid != 0)
        def _():
            pltpu.sync_copy(acc.at[pl.ds(ibase, rows_per_tile)],
                            p1_hbm.at[pl.ds(ibase, rows_per_tile)])

    return agg(x, src_p, dst_p, init0, init1)


def _mlp_bn_body(p0, p1, w1, b1, w2, b2, w3, b3, gamma, beta, out):
    n = out.shape[0]
    h = p0[pl.ds(0, n), :] + p1[pl.ds(0, n), :]
    a1 = jax.nn.sigmoid(
        jnp.dot(h, w1[...], preferred_element_type=jnp.float32) + b1[...])
    a2 = jax.nn.sigmoid(
        jnp.dot(a1, w2[...], preferred_element_type=jnp.float32) + b2[...])
    a3 = jnp.dot(a2, w3[...], preferred_element_type=jnp.float32) + b3[...]
    act = jnp.where(a3 >= 0.0, a3, 0.01 * a3)
    mean = jnp.sum(act, axis=0, keepdims=True) / n
    cent = act - mean
    var = jnp.sum(cent * cent, axis=0, keepdims=True) / n
    out[...] = cent * lax.rsqrt(var + 1e-5) * gamma[...] + beta[...]


def kernel(x, edge_index, W1, b1, W2, b2, W3, b3, gamma, beta):
    n, d = x.shape
    e = edge_index.shape[1]

    chunks = -(-e // L)
    cpt = IB * (-(-chunks // (IB * NW)))
    e_pad = NW * cpt * L
    n_pad = (-(-(n + 1) // 128)) * 128

    src = edge_index[0]
    dst = edge_index[1]
    pad = e_pad - e
    iota = jnp.arange(pad, dtype=jnp.int32)
    src_p = jnp.concatenate([src, iota % n]).reshape(NW, cpt, L)
    dst_p = jnp.concatenate([dst, n + iota % (n_pad - n)]).reshape(NW, cpt, L)
    init0 = jnp.concatenate([x, jnp.zeros((n_pad - n, d), jnp.float32)], axis=0)
    init1 = jnp.zeros((n_pad, d), jnp.float32)

    p0, p1 = _sc_aggregate(n_pad, cpt, x, src_p, dst_p, init0, init1)

    h = pl.pallas_call(
        _mlp_bn_body,
        out_shape=jax.ShapeDtypeStruct((n, d), jnp.float32),
    )(p0, p1, W1, b1.reshape(1, -1), W2, b2.reshape(1, -1),
      W3, b3.reshape(1, -1), gamma.reshape(1, -1), beta.reshape(1, -1))

    return (h, edge_index)


# speedup vs baseline: 3.2253x; 1.0200x over previous
import functools

import jax
import jax.numpy as jnp
from jax import lax
from jax.experimental import pallas as pl
from jax.experimental.pallas import tpu as pltpu
from jax.experimental.pallas import tpu_sc as plsc

L = 128
NW = 32
NS = 16
IB = 40
NB = 2


def _sc_aggregate(n_pad, cpt, x, src_p, dst_p, zslab):
    n = x.shape[0]
    d = x.shape[1]
    rows_per_tile = n_pad // NS
    last_full = (n // rows_per_tile) * rows_per_tile
    tail_x = n - last_full
    tail_z = n_pad - n

    mesh = plsc.VectorSubcoreMesh(core_axis_name="c", subcore_axis_name="s")

    @functools.partial(
        pl.kernel,
        out_type=(
            jax.ShapeDtypeStruct((n_pad, d), jnp.float32),
            jax.ShapeDtypeStruct((n_pad, d), jnp.float32),
        ),
        mesh=mesh,
        scratch_types=[
            pltpu.VMEM_SHARED((n_pad, d), jnp.float32),
            pltpu.VMEM((IB, L), jnp.int32),
            pltpu.VMEM((IB, L), jnp.int32),
        ]
        + [pltpu.VMEM((L, d), jnp.float32)] * NB
        + [pltpu.SemaphoreType.DMA] * (2 * NB),
    )
    def agg(x_hbm, src_hbm, dst_hbm, zslab_hbm, p0_hbm, p1_hbm,
            acc, src_idx, dst_idx, *bufs_sems):
        rows = bufs_sems[:NB]
        gsem = bufs_sems[NB:2 * NB]
        ssem = bufs_sems[2 * NB:]
        cid = lax.axis_index("c")
        sid = lax.axis_index("s")
        wid = sid * 2 + cid

        ibase = sid * rows_per_tile

        @pl.when((cid == 0) & (ibase + rows_per_tile <= last_full))
        def _():
            pltpu.sync_copy(x_hbm.at[pl.ds(ibase, rows_per_tile)],
                            acc.at[pl.ds(ibase, rows_per_tile)])

        @pl.when((cid == 0) & (ibase + rows_per_tile > last_full))
        def _():
            if tail_x:
                pltpu.sync_copy(x_hbm.at[pl.ds(last_full, tail_x)],
                                acc.at[pl.ds(last_full, tail_x)])
            pltpu.sync_copy(zslab_hbm.at[pl.ds(0, tail_z)],
                            acc.at[pl.ds(n, tail_z)])

        @pl.when(cid != 0)
        def _():
            pltpu.sync_copy(zslab_hbm,
                            acc.at[pl.ds(ibase, rows_per_tile)])

        plsc.subcore_barrier()

        def gissue(j, b):
            pltpu.async_copy(x_hbm.at[src_idx.at[j]], rows[b], gsem[b])

        def gwait(b):
            pltpu.make_async_copy(
                x_hbm.at[src_idx.at[0]], rows[b], gsem[b]).wait()

        def sissue(j, b):
            pltpu.async_copy(rows[b], acc.at[dst_idx.at[j]], ssem[b],
                             add=True)

        def swait(b):
            pltpu.make_async_copy(
                rows[b], acc.at[dst_idx.at[0]], ssem[b]).wait()

        def blk_body(bi, carry):
            pltpu.sync_copy(src_hbm.at[wid, pl.ds(bi * IB, IB)], src_idx)
            pltpu.sync_copy(dst_hbm.at[wid, pl.ds(bi * IB, IB)], dst_idx)
            for b in range(NB):
                gissue(b, b)

            def grp(g, c):
                j0 = g * NB
                for b in range(NB):
                    gwait(b)
                    sissue(j0 + b, b)
                for b in range(NB):
                    @pl.when(j0 + NB + b < IB)
                    def _(b=b):
                        swait(b)
                        gissue(j0 + NB + b, b)
                return c

            carry = lax.fori_loop(0, IB // NB, grp, carry)
            for b in range(NB):
                swait(b)
            return carry

        lax.fori_loop(0, cpt // IB, blk_body, 0)
        plsc.subcore_barrier()

        @pl.when(cid == 0)
        def _():
            pltpu.sync_copy(acc.at[pl.ds(ibase, rows_per_tile)],
                            p0_hbm.at[pl.ds(ibase, rows_per_tile)])

        @pl.when(cid != 0)
        def _():
            pltpu.sync_copy(acc.at[pl.ds(ibase, rows_per_tile)],
                            p1_hbm.at[pl.ds(ibase, rows_per_tile)])

    return agg(x, src_p, dst_p, zslab)


def _mlp_bn_body(p0, p1, w1, b1, w2, b2, w3, b3, gamma, beta, out):
    n = out.shape[0]
    h = p0[pl.ds(0, n), :] + p1[pl.ds(0, n), :]
    a1 = jax.nn.sigmoid(
        jnp.dot(h, w1[...], preferred_element_type=jnp.float32) + b1[...])
    a2 = jax.nn.sigmoid(
        jnp.dot(a1, w2[...], preferred_element_type=jnp.float32) + b2[...])
    a3 = jnp.dot(a2, w3[...], preferred_element_type=jnp.float32) + b3[...]
    act = jnp.where(a3 >= 0.0, a3, 0.01 * a3)
    mean = jnp.sum(act, axis=0, keepdims=True) / n
    cent = act - mean
    var = jnp.sum(cent * cent, axis=0, keepdims=True) / n
    out[...] = cent * lax.rsqrt(var + 1e-5) * gamma[...] + beta[...]


def kernel(x, edge_index, W1, b1, W2, b2, W3, b3, gamma, beta):
    n, d = x.shape
    e = edge_index.shape[1]

    chunks = -(-e // L)
    cpt = IB * (-(-chunks // (IB * NW)))
    e_pad = NW * cpt * L
    n_pad = (-(-(n + 1) // 128)) * 128

    src = edge_index[0]
    dst = edge_index[1]
    pad = e_pad - e
    iota = jnp.arange(pad, dtype=jnp.int32)
    src_p = jnp.concatenate([src, iota % n]).reshape(NW, cpt, L)
    dst_p = jnp.concatenate([dst, n + iota % (n_pad - n)]).reshape(NW, cpt, L)
    zslab = jnp.zeros((n_pad // NS, d), jnp.float32)

    p0, p1 = _sc_aggregate(n_pad, cpt, x, src_p, dst_p, zslab)

    h = pl.pallas_call(
        _mlp_bn_body,
        out_shape=jax.ShapeDtypeStruct((n, d), jnp.float32),
    )(p0, p1, W1, b1.reshape(1, -1), W2, b2.reshape(1, -1),
      W3, b3.reshape(1, -1), gamma.reshape(1, -1), beta.reshape(1, -1))

    return (h, edge_index)


# speedup vs baseline: 3.3653x; 1.0434x over previous
import functools

import jax
import jax.numpy as jnp
from jax import lax
from jax.experimental import pallas as pl
from jax.experimental.pallas import tpu as pltpu
from jax.experimental.pallas import tpu_sc as plsc

L = 128
NW = 32
NS = 16
IB = 20
NB = 2


def _sc_aggregate(n_pad, n_blocks_total, q_blocks, x, edges, zslab):
    n = x.shape[0]
    d = x.shape[1]
    rows_per_tile = n_pad // NS
    last_full = (n // rows_per_tile) * rows_per_tile
    tail_x = n - last_full
    tail_z = n_pad - n

    mesh = plsc.VectorSubcoreMesh(core_axis_name="c", subcore_axis_name="s")

    @functools.partial(
        pl.kernel,
        out_type=(
            jax.ShapeDtypeStruct((n_pad, d), jnp.float32),
            jax.ShapeDtypeStruct((n_pad, d), jnp.float32),
        ),
        mesh=mesh,
        scratch_types=[
            pltpu.VMEM_SHARED((n_pad, d), jnp.float32),
            pltpu.VMEM((IB * L,), jnp.int32),
            pltpu.VMEM((IB * L,), jnp.int32),
            pltpu.VMEM((NB, L), jnp.int32),
        ]
        + [pltpu.VMEM((L, d), jnp.float32)] * NB
        + [pltpu.SemaphoreType.DMA] * (2 * NB),
    )
    def agg(x_hbm, edge_hbm, zslab_hbm, p0_hbm, p1_hbm,
            acc, src_idx, dst_idx, dst_row, *bufs_sems):
        rows = bufs_sems[:NB]
        gsem = bufs_sems[NB:2 * NB]
        ssem = bufs_sems[2 * NB:]
        cid = lax.axis_index("c")
        sid = lax.axis_index("s")
        wid = sid * 2 + cid
        block_off = wid * q_blocks
        n_blocks = jnp.minimum(
            q_blocks, jnp.maximum(0, n_blocks_total - block_off))

        ibase = sid * rows_per_tile

        @pl.when((cid == 0) & (ibase + rows_per_tile <= last_full))
        def _():
            pltpu.sync_copy(x_hbm.at[pl.ds(ibase, rows_per_tile)],
                            acc.at[pl.ds(ibase, rows_per_tile)])

        @pl.when((cid == 0) & (ibase + rows_per_tile > last_full))
        def _():
            if tail_x:
                pltpu.sync_copy(x_hbm.at[pl.ds(last_full, tail_x)],
                                acc.at[pl.ds(last_full, tail_x)])
            pltpu.sync_copy(zslab_hbm.at[pl.ds(0, tail_z)],
                            acc.at[pl.ds(n, tail_z)])

        @pl.when(cid != 0)
        def _():
            pltpu.sync_copy(zslab_hbm,
                            acc.at[pl.ds(ibase, rows_per_tile)])

        plsc.subcore_barrier()

        def gissue(j, b):
            pltpu.async_copy(
                x_hbm.at[src_idx.at[pl.ds(j * L, L)]], rows[b], gsem[b])

        def gwait(b):
            pltpu.make_async_copy(
                x_hbm.at[src_idx.at[pl.ds(0, L)]], rows[b], gsem[b]).wait()

        def sissue(j, b):
            for k in range(L // 16):
                dst_row[b, pl.ds(k * 16, 16)] = (
                    dst_idx[pl.ds(j * L + k * 16, 16)])
            pltpu.async_copy(rows[b], acc.at[dst_row.at[b]], ssem[b],
                             add=True)

        def swait(b):
            pltpu.make_async_copy(
                rows[b], acc.at[dst_row.at[b]], ssem[b]).wait()

        def blk_body(bi, carry):
            ebase = (block_off + bi) * (IB * L)
            pltpu.sync_copy(edge_hbm.at[0, pl.ds(ebase, IB * L)], src_idx)
            pltpu.sync_copy(edge_hbm.at[1, pl.ds(ebase, IB * L)], dst_idx)
            for b in range(NB):
                gissue(b, b)

            def grp(g, c):
                j0 = g * NB
                for b in range(NB):
                    gwait(b)
                    sissue(j0 + b, b)
                for b in range(NB):
                    @pl.when(j0 + NB + b < IB)
                    def _(b=b):
                        swait(b)
                        gissue(j0 + NB + b, b)
                return c

            carry = lax.fori_loop(0, IB // NB, grp, carry)
            for b in range(NB):
                swait(b)
            return carry

        lax.fori_loop(0, n_blocks, blk_body, 0)
        plsc.subcore_barrier()

        @pl.when(cid == 0)
        def _():
            pltpu.sync_copy(acc.at[pl.ds(ibase, rows_per_tile)],
                            p0_hbm.at[pl.ds(ibase, rows_per_tile)])

        @pl.when(cid != 0)
        def _():
            pltpu.sync_copy(acc.at[pl.ds(ibase, rows_per_tile)],
                            p1_hbm.at[pl.ds(ibase, rows_per_tile)])

    return agg(x, edges, zslab)


def _mlp_bn_body(p0, p1, w1, b1, w2, b2, w3, b3, gamma, beta, out):
    n = out.shape[0]
    h = p0[pl.ds(0, n), :] + p1[pl.ds(0, n), :]
    a1 = jax.nn.sigmoid(
        jnp.dot(h, w1[...], preferred_element_type=jnp.float32) + b1[...])
    a2 = jax.nn.sigmoid(
        jnp.dot(a1, w2[...], preferred_element_type=jnp.float32) + b2[...])
    a3 = jnp.dot(a2, w3[...], preferred_element_type=jnp.float32) + b3[...]
    act = jnp.where(a3 >= 0.0, a3, 0.01 * a3)
    mean = jnp.sum(act, axis=0, keepdims=True) / n
    cent = act - mean
    var = jnp.sum(cent * cent, axis=0, keepdims=True) / n
    out[...] = cent * lax.rsqrt(var + 1e-5) * gamma[...] + beta[...]


def kernel(x, edge_index, W1, b1, W2, b2, W3, b3, gamma, beta):
    n, d = x.shape
    e = edge_index.shape[1]

    eb = IB * L
    n_blocks_total = -(-e // eb)
    q_blocks = -(-n_blocks_total // NW)
    e_pad = n_blocks_total * eb
    n_pad = (-(-(n + 1) // 128)) * 128

    if e_pad > e:
        iota = jnp.arange(e_pad - e, dtype=jnp.int32)
        edges = jnp.concatenate(
            [edge_index,
             jnp.stack([iota % n, n + iota % (n_pad - n)])], axis=1)
    else:
        edges = edge_index

    zslab = jnp.zeros((n_pad // NS, d), jnp.float32)

    p0, p1 = _sc_aggregate(n_pad, n_blocks_total, q_blocks, x, edges, zslab)

    h = pl.pallas_call(
        _mlp_bn_body,
        out_shape=jax.ShapeDtypeStruct((n, d), jnp.float32),
    )(p0, p1, W1, b1.reshape(1, -1), W2, b2.reshape(1, -1),
      W3, b3.reshape(1, -1), gamma.reshape(1, -1), beta.reshape(1, -1))

    return (h, edge_index)


# speedup vs baseline: 3.3761x; 1.0032x over previous
import functools

import jax
import jax.numpy as jnp
from jax import lax
from jax.experimental import pallas as pl
from jax.experimental.pallas import tpu as pltpu
from jax.experimental.pallas import tpu_sc as plsc

L = 128
NW = 32
NS = 16
IB = 40
NB = 2


def _sc_aggregate(n_pad, n_blocks_total, q_blocks, x, edges, zslab):
    n = x.shape[0]
    d = x.shape[1]
    rows_per_tile = n_pad // NS
    last_full = (n // rows_per_tile) * rows_per_tile
    tail_x = n - last_full
    tail_z = n_pad - n

    mesh = plsc.VectorSubcoreMesh(core_axis_name="c", subcore_axis_name="s")

    @functools.partial(
        pl.kernel,
        out_type=(
            jax.ShapeDtypeStruct((n_pad, d), jnp.float32),
            jax.ShapeDtypeStruct((n_pad, d), jnp.float32),
        ),
        mesh=mesh,
        scratch_types=[
            pltpu.VMEM_SHARED((n_pad, d), jnp.float32),
            pltpu.VMEM((IB * L,), jnp.int32),
            pltpu.VMEM((IB * L,), jnp.int32),
            pltpu.VMEM((NB, L), jnp.int32),
        ]
        + [pltpu.VMEM((L, d), jnp.float32)] * NB
        + [pltpu.SemaphoreType.DMA] * (2 * NB),
    )
    def agg(x_hbm, edge_hbm, zslab_hbm, p0_hbm, p1_hbm,
            acc, src_idx, dst_idx, dst_row, *bufs_sems):
        rows = bufs_sems[:NB]
        gsem = bufs_sems[NB:2 * NB]
        ssem = bufs_sems[2 * NB:]
        cid = lax.axis_index("c")
        sid = lax.axis_index("s")
        wid = sid * 2 + cid
        block_off = wid * q_blocks
        n_blocks = jnp.minimum(
            q_blocks, jnp.maximum(0, n_blocks_total - block_off))

        ibase = sid * rows_per_tile

        @pl.when((cid == 0) & (ibase + rows_per_tile <= last_full))
        def _():
            pltpu.sync_copy(x_hbm.at[pl.ds(ibase, rows_per_tile)],
                            acc.at[pl.ds(ibase, rows_per_tile)])

        @pl.when((cid == 0) & (ibase + rows_per_tile > last_full))
        def _():
            if tail_x:
                pltpu.sync_copy(x_hbm.at[pl.ds(last_full, tail_x)],
                                acc.at[pl.ds(last_full, tail_x)])
            pltpu.sync_copy(zslab_hbm.at[pl.ds(0, tail_z)],
                            acc.at[pl.ds(n, tail_z)])

        @pl.when(cid != 0)
        def _():
            pltpu.sync_copy(zslab_hbm,
                            acc.at[pl.ds(ibase, rows_per_tile)])

        plsc.subcore_barrier()

        def gissue(j, b):
            pltpu.async_copy(
                x_hbm.at[src_idx.at[pl.ds(j * L, L)]], rows[b], gsem[b])

        def gwait(b):
            pltpu.make_async_copy(
                x_hbm.at[src_idx.at[pl.ds(0, L)]], rows[b], gsem[b]).wait()

        def sissue(j, b):
            for k in range(L // 16):
                dst_row[b, pl.ds(k * 16, 16)] = (
                    dst_idx[pl.ds(j * L + k * 16, 16)])
            pltpu.async_copy(rows[b], acc.at[dst_row.at[b]], ssem[b],
                             add=True)

        def swait(b):
            pltpu.make_async_copy(
                rows[b], acc.at[dst_row.at[b]], ssem[b]).wait()

        def blk_body(bi, carry):
            ebase = (block_off + bi) * (IB * L)
            pltpu.sync_copy(edge_hbm.at[0, pl.ds(ebase, IB * L)], src_idx)
            pltpu.sync_copy(edge_hbm.at[1, pl.ds(ebase, IB * L)], dst_idx)
            for b in range(NB):
                gissue(b, b)

            def grp(g, c):
                j0 = g * NB
                for b in range(NB):
                    gwait(b)
                    sissue(j0 + b, b)
                for b in range(NB):
                    @pl.when(j0 + NB + b < IB)
                    def _(b=b):
                        swait(b)
                        gissue(j0 + NB + b, b)
                return c

            carry = lax.fori_loop(0, IB // NB, grp, carry)
            for b in range(NB):
                swait(b)
            return carry

        lax.fori_loop(0, n_blocks, blk_body, 0)
        plsc.subcore_barrier()

        @pl.when(cid == 0)
        def _():
            pltpu.sync_copy(acc.at[pl.ds(ibase, rows_per_tile)],
                            p0_hbm.at[pl.ds(ibase, rows_per_tile)])

        @pl.when(cid != 0)
        def _():
            pltpu.sync_copy(acc.at[pl.ds(ibase, rows_per_tile)],
                            p1_hbm.at[pl.ds(ibase, rows_per_tile)])

    return agg(x, edges, zslab)


def _mlp_bn_body(p0, p1, w1, b1, w2, b2, w3, b3, gamma, beta, out):
    n = out.shape[0]
    h = p0[pl.ds(0, n), :] + p1[pl.ds(0, n), :]
    a1 = jax.nn.sigmoid(
        jnp.dot(h, w1[...], preferred_element_type=jnp.float32) + b1[...])
    a2 = jax.nn.sigmoid(
        jnp.dot(a1, w2[...], preferred_element_type=jnp.float32) + b2[...])
    a3 = jnp.dot(a2, w3[...], preferred_element_type=jnp.float32) + b3[...]
    act = jnp.where(a3 >= 0.0, a3, 0.01 * a3)
    mean = jnp.sum(act, axis=0, keepdims=True) / n
    cent = act - mean
    var = jnp.sum(cent * cent, axis=0, keepdims=True) / n
    out[...] = cent * lax.rsqrt(var + 1e-5) * gamma[...] + beta[...]


def kernel(x, edge_index, W1, b1, W2, b2, W3, b3, gamma, beta):
    n, d = x.shape
    e = edge_index.shape[1]

    eb = IB * L
    n_blocks_total = -(-e // eb)
    q_blocks = -(-n_blocks_total // NW)
    e_pad = n_blocks_total * eb
    n_pad = (-(-(n + 1) // 128)) * 128

    if e_pad > e:
        iota = jnp.arange(e_pad - e, dtype=jnp.int32)
        edges = jnp.concatenate(
            [edge_index,
             jnp.stack([iota % n, n + iota % (n_pad - n)])], axis=1)
    else:
        edges = edge_index

    zslab = jnp.zeros((n_pad // NS, d), jnp.float32)

    p0, p1 = _sc_aggregate(n_pad, n_blocks_total, q_blocks, x, edges, zslab)

    h = pl.pallas_call(
        _mlp_bn_body,
        out_shape=jax.ShapeDtypeStruct((n, d), jnp.float32),
    )(p0, p1, W1, b1.reshape(1, -1), W2, b2.reshape(1, -1),
      W3, b3.reshape(1, -1), gamma.reshape(1, -1), beta.reshape(1, -1))

    return (h, edge_index)
